# Initial kernel scaffold; baseline (speedup 1.0000x reference)
import jax
import jax.numpy as jnp
from jax.experimental import pallas as pl


def kernel(token_ids, weight):
    raise NotImplementedError("write your pallas kernel here")


# speedup vs baseline: 1.0238x; 1.0238x over previous
import functools

import jax
import jax.numpy as jnp
from jax import lax
from jax.experimental import pallas as pl
from jax.experimental.pallas import tpu as pltpu
from jax.experimental.pallas import tpu_sc as plsc

_NW = 32
_CHUNK = 128


@functools.partial(jax.jit, static_argnums=(2, 3))
def _lookup(idx2d, weight, total, n_chunks):
    dim = weight.shape[1]

    @functools.partial(
        pl.kernel,
        mesh=plsc.VectorSubcoreMesh(core_axis_name="c", subcore_axis_name="s"),
        out_type=jax.ShapeDtypeStruct((total, dim), jnp.float32),
        scratch_types=[
            pltpu.VMEM((n_chunks, _CHUNK), jnp.int32),
            pltpu.VMEM((_CHUNK, dim), jnp.float32),
            pltpu.SemaphoreType.DMA,
        ],
        compiler_params=pltpu.CompilerParams(use_tc_tiling_on_sc=False),
    )
    def k(idx_hbm, table_hbm, out_hbm, idx_v, buf, sem):
        wid = lax.axis_index("s") * 2 + lax.axis_index("c")
        row0 = wid * n_chunks
        base = row0 * _CHUNK
        pltpu.sync_copy(idx_hbm.at[pl.ds(row0, n_chunks)], idx_v)

        def body(j, carry):
            pltpu.async_copy(table_hbm.at[idx_v.at[j]], buf, sem).wait()
            pltpu.sync_copy(buf, out_hbm.at[pl.ds(base + j * _CHUNK, _CHUNK)])
            return carry

        lax.fori_loop(0, n_chunks, body, 0)

    return k(idx2d, weight)


def kernel(token_ids, weight):
    b, s = token_ids.shape
    dim = weight.shape[1]
    total = b * s
    idx2d = token_ids.reshape(total // _CHUNK, _CHUNK).astype(jnp.int32)
    n_chunks = total // (_NW * _CHUNK)
    out = _lookup(idx2d, weight, total, n_chunks)
    return out.reshape(b, s, dim)


# speedup vs baseline: 1.2707x; 1.2412x over previous
import functools

import jax
import jax.numpy as jnp
from jax import lax
from jax.experimental import pallas as pl
from jax.experimental.pallas import tpu as pltpu
from jax.experimental.pallas import tpu_sc as plsc

_NW = 32
_IDXCH = 64
_GRP = 2


@jax.jit
def _lookup(token_ids, table2):
    nb, seq = token_ids.shape
    nblk, blkw = table2.shape
    dim = 32
    nb_per_w = nb // _NW

    @functools.partial(
        pl.kernel,
        mesh=plsc.VectorSubcoreMesh(core_axis_name="c", subcore_axis_name="s"),
        out_type=jax.ShapeDtypeStruct((nb, seq, dim), jnp.float32),
        scratch_types=[
            pltpu.VMEM((_IDXCH, seq), jnp.int32),
            pltpu.VMEM((_IDXCH, 64), jnp.int32),
            pltpu.VMEM((seq, blkw), jnp.float32),
            pltpu.VMEM((seq, blkw), jnp.float32),
            pltpu.VMEM((_GRP, seq, dim), jnp.float32),
            pltpu.SemaphoreType.DMA,
            pltpu.SemaphoreType.DMA,
        ],
    )
    def k(idx_hbm, table_hbm, out_hbm, idx_v, blk_v, buf_a, buf_b, out_v, sem_a, sem_b):
        wid = lax.axis_index("s") * 2 + lax.axis_index("c")
        b_base = wid * nb_per_w

        def idx_chunk(ci, carry):
            b0 = b_base + ci * _IDXCH
            pltpu.sync_copy(idx_hbm.at[pl.ds(b0, _IDXCH)], idx_v)
            for off in (0, 16, 32, 34):
                for r in range(_IDXCH):
                    blk_v[r, pl.ds(off, 16)] = idx_v[r, pl.ds(off, 16)] >> 2

            def group(gi, c2):
                r0 = gi * _GRP
                ga = pltpu.async_copy(
                    table_hbm.at[blk_v.at[r0, pl.ds(0, seq)]], buf_a, sem_a)
                gb = pltpu.async_copy(
                    table_hbm.at[blk_v.at[r0 + 1, pl.ds(0, seq)]], buf_b, sem_b)
                ga.wait()
                gb.wait()
                for g in range(_GRP):
                    buf = buf_a if g == 0 else buf_b
                    for t0 in (0, 16, 32, 34):
                        offv = (idx_v[r0 + g, pl.ds(t0, 16)] & 3) * dim
                        for lane in range(14 if t0 == 34 else 0, 16):
                            t = t0 + lane
                            off = offv[lane]
                            out_v[g, t, pl.ds(0, 16)] = buf[t, pl.ds(off, 16)]
                            out_v[g, t, pl.ds(16, 16)] = buf[t, pl.ds(off + 16, 16)]
                pltpu.sync_copy(out_v, out_hbm.at[pl.ds(b0 + r0, _GRP)])
                return c2

            lax.fori_loop(0, _IDXCH // _GRP, group, 0)
            return carry

        lax.fori_loop(0, nb_per_w // _IDXCH, idx_chunk, 0)

    return k(token_ids, table2)


def kernel(token_ids, weight):
    table2 = weight.reshape(weight.shape[0] // 4, 128)
    return _lookup(token_ids.astype(jnp.int32), table2)


# speedup vs baseline: 1.3829x; 1.0883x over previous
import functools

import jax
import jax.numpy as jnp
from jax import lax
from jax.experimental import pallas as pl
from jax.experimental.pallas import tpu as pltpu
from jax.experimental.pallas import tpu_sc as plsc

_NW = 32
_IDXCH = 64
_GRP = 4
_NGRP = _IDXCH // _GRP
_ROW = 104


@jax.jit
def _lookup(token_ids, table2):
    nb, seq = token_ids.shape
    dim = 32
    nb_per_w = nb // _NW
    nchunks = nb_per_w // _IDXCH

    @functools.partial(
        pl.kernel,
        mesh=plsc.VectorSubcoreMesh(core_axis_name="c", subcore_axis_name="s"),
        out_type=jax.ShapeDtypeStruct((nb, seq, dim), jnp.float32),
        scratch_types=[
            pltpu.VMEM((_IDXCH, seq), jnp.int32),
            pltpu.VMEM((2 * _NGRP, _ROW), jnp.int32),
            pltpu.VMEM((2 * seq, 128), jnp.float32),
            pltpu.VMEM((2 * seq, 128), jnp.float32),
            pltpu.VMEM((2 * seq, 128), jnp.float32),
            pltpu.VMEM((2 * seq, 128), jnp.float32),
            pltpu.VMEM((2 * _GRP, seq, dim), jnp.float32),
            pltpu.SemaphoreType.DMA,
            pltpu.SemaphoreType.DMA,
            pltpu.SemaphoreType.DMA,
        ],
    )
    def k(idx_hbm, table_hbm, out_hbm, idx_v, blk_v,
          buf_a0, buf_a1, buf_b0, buf_b1, ov,
          sem_a, sem_b, sem_o):
        wid = lax.axis_index("s") * 2 + lax.axis_index("c")
        b_base = wid * nb_per_w

        def fire(gi, buf0, buf1, sem):
            c0 = pltpu.async_copy(
                table_hbm.at[blk_v.at[2 * gi, pl.ds(0, 2 * seq)]], buf0, sem)
            c1 = pltpu.async_copy(
                table_hbm.at[blk_v.at[2 * gi + 1, pl.ds(0, 2 * seq)]], buf1, sem)
            return c0, c1

        def extract(gi, buf0, buf1, half):
            r0 = gi * _GRP
            for lr in range(_GRP):
                buf = buf0 if lr < 2 else buf1
                tbase = seq * (lr % 2)
                for t0 in (0, 16, 32, 34):
                    offv = (idx_v[r0 + lr, pl.ds(t0, 16)] & 3) * dim
                    for lane in range(14 if t0 == 34 else 0, 16):
                        s = t0 + lane
                        off = offv[lane]
                        tloc = tbase + s
                        for h in (0, 1):
                            ov[half * _GRP + lr, s, pl.ds(16 * h, 16)] = (
                                buf[tloc, pl.ds(off + 16 * h, 16)])

        def chunk(ci, carry):
            b0 = pl.multiple_of(b_base + ci * _IDXCH, _IDXCH)
            pltpu.sync_copy(idx_hbm.at[pl.ds(b0, _IDXCH)], idx_v)
            for r in range(_IDXCH):
                gi, p = r // _GRP, r % _GRP
                row, dbase = 2 * gi + p // 2, seq * (p % 2)
                for off in (0, 16, 32, 34):
                    blk_v[row, pl.ds(dbase + off, 16)] = (
                        idx_v[r, pl.ds(off, 16)] >> 2)

            def group_pair(kk, c2):
                ga = kk * 2
                a0, a1 = fire(ga, buf_a0, buf_a1, sem_a)
                b0_, b1_ = fire(ga + 1, buf_b0, buf_b1, sem_b)
                a0.wait()
                a1.wait()
                extract(ga, buf_a0, buf_a1, 0)
                b0_.wait()
                b1_.wait()
                extract(ga + 1, buf_b0, buf_b1, 1)
                pltpu.sync_copy(
                    ov, out_hbm.at[pl.ds(
                        pl.multiple_of(b0 + kk * 2 * _GRP, 2 * _GRP),
                        2 * _GRP)])
                return c2

            lax.fori_loop(0, _NGRP // 2, group_pair, 0)
            return carry

        lax.fori_loop(0, nchunks, chunk, 0)

    return k(token_ids, table2)


def kernel(token_ids, weight):
    table2 = weight.reshape(weight.shape[0] // 4, 128)
    return _lookup(token_ids.astype(jnp.int32), table2)
